# baseline (speedup 1.0000x reference)
import functools

import jax
import jax.numpy as jnp
from jax import lax
from jax.experimental import pallas as pl
from jax.experimental.pallas import tpu as pltpu
from jax.experimental.pallas import tpu_sc as plsc

_B, _C, _H, _W = 8, 16, 512, 512
_HW = _H * _W
_NPIX = _B * _HW
_NW = 32
_NROWS = _B * _H
_ROWS_PER_W = _NROWS // _NW
_CH = _W
_L = 16
_G = _CH // _L
_ISL = 128
_NSL = 4 * _CH // _ISL


def _body(table, fx_hbm, fy_hbm, out_hbm, fx_v, fy_v, idx_v, w_v, rows_v,
          out_t, sem):
    cid = lax.axis_index("c")
    sid = lax.axis_index("s")
    wid = cid * 16 + sid
    lane = jnp.arange(_L, dtype=jnp.int32)

    def chunk_body(t, carry):
        row = wid * _ROWS_PER_W + t
        b = row >> 9
        h = row & (_H - 1)
        base_pix = row * _W
        pltpu.sync_copy(fx_hbm.at[b, h], fx_v)
        pltpu.sync_copy(fy_hbm.at[b, h], fy_v)

        def gen_body(g, carry2):
            ww = g * _L + lane
            fx = fx_v[pl.ds(g * _L, _L)]
            fy = fy_v[pl.ds(g * _L, _L)]
            x = ww.astype(jnp.float32) + fx
            y = h.astype(jnp.float32) + fy
            x = jnp.minimum(jnp.maximum(x, -4.0), float(_W) + 4.0)
            y = jnp.minimum(jnp.maximum(y, -4.0), float(_H) + 4.0)
            xt = x.astype(jnp.int32)
            x0 = jnp.where(xt.astype(jnp.float32) > x, xt - 1, xt)
            yt = y.astype(jnp.int32)
            y0 = jnp.where(yt.astype(jnp.float32) > y, yt - 1, yt)
            dx = x - x0.astype(jnp.float32)
            dy = y - y0.astype(jnp.float32)
            one = jnp.float32(1.0)
            zero = jnp.float32(0.0)
            vx0 = jnp.where((x0 >= 0) & (x0 <= _W - 1), one, zero)
            vx1 = jnp.where((x0 >= -1) & (x0 <= _W - 2), one, zero)
            vy0 = jnp.where((y0 >= 0) & (y0 <= _H - 1), one, zero)
            vy1 = jnp.where((y0 >= -1) & (y0 <= _H - 2), one, zero)
            cx0 = jnp.minimum(jnp.maximum(x0, 0), _W - 1)
            cx1 = jnp.minimum(jnp.maximum(x0 + 1, 0), _W - 1)
            cy0 = jnp.minimum(jnp.maximum(y0, 0), _H - 1) << 9
            cy1 = jnp.minimum(jnp.maximum(y0 + 1, 0), _H - 1) << 9
            brow = b << 18

            col = (g & 7) * _L
            r = g >> 3
            idx_v[r, pl.ds(col, _L)] = brow + cy0 + cx0
            idx_v[r + 4, pl.ds(col, _L)] = brow + cy0 + cx1
            idx_v[r + 8, pl.ds(col, _L)] = brow + cy1 + cx0
            idx_v[r + 12, pl.ds(col, _L)] = brow + cy1 + cx1

            omdx = one - dx
            omdy = one - dy
            s = pl.ds(g * _L, _L)
            w_v[0, s] = omdx * omdy * (vx0 * vy0)
            w_v[1, s] = dx * omdy * (vx1 * vy0)
            w_v[2, s] = omdx * dy * (vx0 * vy1)
            w_v[3, s] = dx * dy * (vx1 * vy1)
            return carry2

        lax.fori_loop(0, _G, gen_body, 0)

        copies = []
        for k in range(_NSL):
            copies.append(pltpu.async_copy(
                table.at[idx_v.at[k]],
                rows_v.at[pl.ds(k * _ISL, _ISL)],
                sem,
            ))
        for c in copies:
            c.wait()

        def blend_body(g, carry2):
            s = pl.ds(g * _L, _L)
            wa = w_v[0, s]
            wb = w_v[1, s]
            wc = w_v[2, s]
            wd = w_v[3, s]
            zero = jnp.float32(0.0)
            for j in range(_L):
                onehot = lane == j
                was = jnp.sum(jnp.where(onehot, wa, zero))
                wbs = jnp.sum(jnp.where(onehot, wb, zero))
                wcs = jnp.sum(jnp.where(onehot, wc, zero))
                wds = jnp.sum(jnp.where(onehot, wd, zero))
                pp = g * _L + j
                ra = rows_v[pp, :]
                rb = rows_v[pp + _CH, :]
                rc = rows_v[pp + 2 * _CH, :]
                rd = rows_v[pp + 3 * _CH, :]
                acc = was * ra + wbs * rb + wcs * rc + wds * rd
                plsc.store_scatter(out_t, [lane, lane * 0 + pp], acc)
            return carry2

        lax.fori_loop(0, _G, blend_body, 0)

        wcopies = []
        for c in range(_C):
            wcopies.append(pltpu.async_copy(
                out_t.at[c], out_hbm.at[b, c, h], sem))
        for c in wcopies:
            c.wait()
        return carry

    lax.fori_loop(0, _ROWS_PER_W, chunk_body, 0)


_warp_sc = pl.kernel(
    _body,
    out_type=jax.ShapeDtypeStruct((_B, _C, _H, _W), jnp.float32),
    mesh=plsc.VectorSubcoreMesh(core_axis_name="c", subcore_axis_name="s"),
    compiler_params=pltpu.CompilerParams(
        needs_layout_passes=False, use_tc_tiling_on_sc=False
    ),
    scratch_types=[
        pltpu.VMEM((_CH,), jnp.float32),
        pltpu.VMEM((_CH,), jnp.float32),
        pltpu.VMEM((_NSL, _ISL), jnp.int32),
        pltpu.VMEM((4, _CH), jnp.float32),
        pltpu.VMEM((4 * _CH, _C), jnp.float32),
        pltpu.VMEM((_C, _CH), jnp.float32),
        pltpu.SemaphoreType.DMA,
    ],
)


def kernel(src, flow):
    table = jnp.transpose(src.reshape(_B, _C, _HW), (0, 2, 1))
    table = table.reshape(_NPIX, _C)
    fx = flow[:, 0, :, :]
    fy = flow[:, 1, :, :]
    return _warp_sc(table, fx, fy)


# speedup vs baseline: 1.2371x; 1.2371x over previous
import functools

import jax
import jax.numpy as jnp
from jax import lax
from jax.experimental import pallas as pl
from jax.experimental.pallas import tpu as pltpu
from jax.experimental.pallas import tpu_sc as plsc

_B, _C, _H, _W = 8, 16, 512, 512
_HW = _H * _W
_NPIX = _B * _HW
_NW = 32
_NBLK = _B * (_H // 8)
_BLK_PER_W = _NBLK // _NW
_BPIX = 8 * _W
_L = 16
_G = _W // _L
_ISL = 128
_NSL = 4 * _W // _ISL
_SCP = pltpu.CompilerParams(
    needs_layout_passes=False, use_tc_tiling_on_sc=False
)
_MESH = plsc.VectorSubcoreMesh(core_axis_name="c", subcore_axis_name="s")


def _transpose_body(src_t, table, in_v, tbl_t, sem):
    wid = lax.axis_index("c") * 16 + lax.axis_index("s")
    lane = jnp.arange(_L, dtype=jnp.int32)

    def blk_body(t, carry):
        blk = wid * _BLK_PER_W + t
        b = blk >> 6
        hh = blk & 63

        cp0 = pltpu.async_copy(src_t.at[b, 0, hh], in_v.at[0], sem)
        prev = cp0
        for c in range(_C):
            if c + 1 < _C:
                nxt = pltpu.async_copy(
                    src_t.at[b, c + 1, hh], in_v.at[(c + 1) & 1], sem)
            prev.wait()

            def ch_body(i, carry2, c=c):
                ww = i >> 6
                r = (i >> 3) & 7
                s = i & 7
                v = in_v[c & 1, pl.ds(i * _L, _L)]
                rowbase = r * _W + (ww << 7) + (s << 4)
                plsc.store_scatter(
                    tbl_t, [rowbase + lane, lane * 0 + c], v)
                return carry2

            lax.fori_loop(0, _BPIX // _L, ch_body, 0)
            if c + 1 < _C:
                prev = nxt
        pltpu.sync_copy(tbl_t, table.at[pl.ds(blk * _BPIX, _BPIX)])
        return carry

    lax.fori_loop(0, _BLK_PER_W, blk_body, 0)


_make_table = pl.kernel(
    _transpose_body,
    out_type=jax.ShapeDtypeStruct((_NPIX, _C), jnp.float32),
    mesh=_MESH,
    compiler_params=_SCP,
    scratch_types=[
        pltpu.VMEM((2, _BPIX), jnp.float32),
        pltpu.VMEM((_BPIX, _C), jnp.float32),
        pltpu.SemaphoreType.DMA,
    ],
)


def _warp_body(table, fx_hbm, fy_hbm, out_t, fx_v, fy_v, idx_v, w_v, rows_v,
               ob_v, sem):
    wid = lax.axis_index("c") * 16 + lax.axis_index("s")
    lane = jnp.arange(_L, dtype=jnp.int32)

    def blk_body(t, carry):
        blk = wid * _BLK_PER_W + t
        b = blk >> 6
        hh = blk & 63
        brow = b << 18

        def row_body(r, carry1):
            h = (hh << 3) + r
            pltpu.sync_copy(fx_hbm.at[b, h], fx_v)
            pltpu.sync_copy(fy_hbm.at[b, h], fy_v)

            def gen_body(g, carry2):
                ww = g * _L + lane
                fx = fx_v[pl.ds(g * _L, _L)]
                fy = fy_v[pl.ds(g * _L, _L)]
                x = ww.astype(jnp.float32) + fx
                y = h.astype(jnp.float32) + fy
                x = jnp.minimum(jnp.maximum(x, -4.0), float(_W) + 4.0)
                y = jnp.minimum(jnp.maximum(y, -4.0), float(_H) + 4.0)
                xt = x.astype(jnp.int32)
                x0 = jnp.where(xt.astype(jnp.float32) > x, xt - 1, xt)
                yt = y.astype(jnp.int32)
                y0 = jnp.where(yt.astype(jnp.float32) > y, yt - 1, yt)
                dx = x - x0.astype(jnp.float32)
                dy = y - y0.astype(jnp.float32)
                one = jnp.float32(1.0)
                zero = jnp.float32(0.0)
                vx0 = jnp.where((x0 >= 0) & (x0 <= _W - 1), one, zero)
                vx1 = jnp.where((x0 >= -1) & (x0 <= _W - 2), one, zero)
                vy0 = jnp.where((y0 >= 0) & (y0 <= _H - 1), one, zero)
                vy1 = jnp.where((y0 >= -1) & (y0 <= _H - 2), one, zero)
                cx0 = jnp.minimum(jnp.maximum(x0, 0), _W - 1)
                cx1 = jnp.minimum(jnp.maximum(x0 + 1, 0), _W - 1)
                cy0 = jnp.minimum(jnp.maximum(y0, 0), _H - 1) << 9
                cy1 = jnp.minimum(jnp.maximum(y0 + 1, 0), _H - 1) << 9

                col = (g & 7) * _L
                rr = g >> 3
                idx_v[rr, pl.ds(col, _L)] = brow + cy0 + cx0
                idx_v[rr + 4, pl.ds(col, _L)] = brow + cy0 + cx1
                idx_v[rr + 8, pl.ds(col, _L)] = brow + cy1 + cx0
                idx_v[rr + 12, pl.ds(col, _L)] = brow + cy1 + cx1

                omdx = one - dx
                omdy = one - dy
                s = pl.ds(g * _L, _L)
                w_v[0, s] = omdx * omdy * (vx0 * vy0)
                w_v[1, s] = dx * omdy * (vx1 * vy0)
                w_v[2, s] = omdx * dy * (vx0 * vy1)
                w_v[3, s] = dx * dy * (vx1 * vy1)
                return carry2

            lax.fori_loop(0, _G, gen_body, 0)

            copies = []
            for k in range(_NSL):
                copies.append(pltpu.async_copy(
                    table.at[idx_v.at[k]],
                    rows_v.at[pl.ds(k * _ISL, _ISL)],
                    sem,
                ))
            for cpy in copies:
                cpy.wait()

            def blend_body(g, carry2):
                s = pl.ds(g * _L, _L)
                wa = w_v[0, s]
                wb = w_v[1, s]
                wc = w_v[2, s]
                wd = w_v[3, s]
                zero = jnp.float32(0.0)
                for j in range(_L):
                    onehot = lane == j
                    was = jnp.sum(jnp.where(onehot, wa, zero))
                    wbs = jnp.sum(jnp.where(onehot, wb, zero))
                    wcs = jnp.sum(jnp.where(onehot, wc, zero))
                    wds = jnp.sum(jnp.where(onehot, wd, zero))
                    pp = g * _L + j
                    ra = rows_v[pp, :]
                    rb = rows_v[pp + _W, :]
                    rc = rows_v[pp + 2 * _W, :]
                    rd = rows_v[pp + 3 * _W, :]
                    acc = was * ra + wbs * rb + wcs * rc + wds * rd
                    ofs = ((pp >> 7) << 10) + (r << 7) + (pp & 127)
                    plsc.store_scatter(ob_v, [lane, lane * 0 + ofs], acc)
                return carry2

            lax.fori_loop(0, _G, blend_body, 0)
            return carry1

        lax.fori_loop(0, 8, row_body, 0)

        wcopies = []
        for c in range(_C):
            wcopies.append(pltpu.async_copy(
                ob_v.at[c], out_t.at[b, c, hh], sem))
        for cpy in wcopies:
            cpy.wait()
        return carry

    lax.fori_loop(0, _BLK_PER_W, blk_body, 0)


_warp_sc = pl.kernel(
    _warp_body,
    out_type=jax.ShapeDtypeStruct((_B, _C, _H // 8, _BPIX), jnp.float32),
    mesh=_MESH,
    compiler_params=_SCP,
    scratch_types=[
        pltpu.VMEM((_W,), jnp.float32),
        pltpu.VMEM((_W,), jnp.float32),
        pltpu.VMEM((_NSL, _ISL), jnp.int32),
        pltpu.VMEM((4, _W), jnp.float32),
        pltpu.VMEM((4 * _W, _C), jnp.float32),
        pltpu.VMEM((_C, _BPIX), jnp.float32),
        pltpu.SemaphoreType.DMA,
    ],
)


def _to_tile_order(a):
    a = a.reshape(_B, _C, _H // 8, 8, _W // 128, 128)
    a = a.transpose(0, 1, 2, 4, 3, 5)
    return a.reshape(_B, _C, _H // 8, _BPIX)


def _from_tile_order(a):
    a = a.reshape(_B, _C, _H // 8, _W // 128, 8, 128)
    a = a.transpose(0, 1, 2, 4, 3, 5)
    return a.reshape(_B, _C, _H, _W)


def kernel(src, flow):
    table = _make_table(_to_tile_order(src))
    fx = flow[:, 0, :, :]
    fy = flow[:, 1, :, :]
    out_t = _warp_sc(table, fx, fy)
    return _from_tile_order(out_t)
